# baseline (speedup 1.0000x reference)
import jax
import jax.numpy as jnp
from jax.experimental import pallas as pl
from jax.experimental.pallas import tpu as pltpu

_K = 512
_D = 256
_G = 8


def _vq_body(x_ref, cb_ref, out_ref, sq_ref, hi_ref, lo_ref):
    @pl.when(pl.program_id(0) == 0)
    def _init():
        cb = cb_ref[...]
        sq_ref[0:1, :] = 2.0 * jnp.sum(cb, axis=0, keepdims=True)
        sq_ref[1:2, :] = jnp.sum(cb * cb, axis=0, keepdims=True)
        cb_top = cb[:_D, :]
        hi = cb_top.astype(jnp.bfloat16)
        hi_ref[...] = hi
        lo_ref[...] = (cb_top - hi.astype(jnp.float32)).astype(jnp.bfloat16)

    x = x_ref[...]
    s2 = sq_ref[0:1, :]
    q = sq_ref[1:2, :]
    dist2 = jnp.float32(_K) * (x * x) - x * s2 + q
    dist = jnp.sqrt(jnp.maximum(dist2, 0.0))
    idx = jnp.argmin(dist, axis=1).astype(jnp.int32)
    iota_d = jax.lax.broadcasted_iota(jnp.int32, dist.shape, 1)
    oh = (iota_d == idx[:, None]).astype(jnp.bfloat16)
    dims = (((1,), (0,)), ((), ()))
    out_ref[...] = (
        jax.lax.dot_general(oh, hi_ref[...], dims,
                            preferred_element_type=jnp.float32)
        + jax.lax.dot_general(oh, lo_ref[...], dims,
                              preferred_element_type=jnp.float32))


def kernel(codes, codebook):
    b, t, d = codes.shape
    n = b * t
    tb = n // _G
    x = codes.reshape(n, d)
    out = pl.pallas_call(
        _vq_body,
        grid=(_G,),
        in_specs=[
            pl.BlockSpec((tb, d), lambda i: (i, 0)),
            pl.BlockSpec((_K, d), lambda i: (0, 0)),
        ],
        out_specs=pl.BlockSpec((tb, d), lambda i: (i, 0)),
        out_shape=jax.ShapeDtypeStruct((n, d), jnp.float32),
        scratch_shapes=[
            pltpu.VMEM((2, d), jnp.float32),
            pltpu.VMEM((_D, d), jnp.bfloat16),
            pltpu.VMEM((_D, d), jnp.bfloat16),
        ],
    )(x, codebook)
    return out.reshape(b, t, d)


# speedup vs baseline: 1.7608x; 1.7608x over previous
import jax
import jax.numpy as jnp
from jax.experimental import pallas as pl
from jax.experimental.pallas import tpu as pltpu

_K = 512
_D = 256
_G = 2


def _vq_body(x_ref, cb_ref, out_ref, sq_ref, hi_ref, lo_ref):
    @pl.when(pl.program_id(0) == 0)
    def _init():
        cb = cb_ref[...]
        sq_ref[0:1, :] = 2.0 * jnp.sum(cb, axis=0, keepdims=True)
        sq_ref[1:2, :] = jnp.sum(cb * cb, axis=0, keepdims=True)
        cb_top = cb[:_D, :]
        hi = cb_top.astype(jnp.bfloat16)
        hi_ref[...] = hi
        lo_ref[...] = (cb_top - hi.astype(jnp.float32)).astype(jnp.bfloat16)

    x = x_ref[...]
    s2 = sq_ref[0:1, :]
    q = sq_ref[1:2, :]
    dist2 = jnp.float32(_K) * (x * x) - x * s2 + q
    dist = jnp.sqrt(jnp.maximum(dist2, 0.0))
    idx = jnp.argmin(dist, axis=1).astype(jnp.int32)
    iota_d = jax.lax.broadcasted_iota(jnp.int32, dist.shape, 1)
    oh = (iota_d == idx[:, None]).astype(jnp.bfloat16)
    dims = (((1,), (0,)), ((), ()))
    out_ref[...] = (
        jax.lax.dot_general(oh, hi_ref[...], dims,
                            preferred_element_type=jnp.float32)
        + jax.lax.dot_general(oh, lo_ref[...], dims,
                              preferred_element_type=jnp.float32))


def kernel(codes, codebook):
    b, t, d = codes.shape
    n = b * t
    tb = n // _G
    x = codes.reshape(n, d)
    out = pl.pallas_call(
        _vq_body,
        grid=(_G,),
        in_specs=[
            pl.BlockSpec((tb, d), lambda i: (i, 0)),
            pl.BlockSpec((_K, d), lambda i: (0, 0)),
        ],
        out_specs=pl.BlockSpec((tb, d), lambda i: (i, 0)),
        out_shape=jax.ShapeDtypeStruct((n, d), jnp.float32),
        scratch_shapes=[
            pltpu.VMEM((2, d), jnp.float32),
            pltpu.VMEM((_D, d), jnp.bfloat16),
            pltpu.VMEM((_D, d), jnp.bfloat16),
        ],
    )(x, codebook)
    return out.reshape(b, t, d)


# speedup vs baseline: 2.0783x; 1.1803x over previous
import jax
import jax.numpy as jnp
from jax.experimental import pallas as pl

_K = 512
_D = 256


def _vq_body(x_ref, cb_ref, out_ref):
    x = x_ref[...]
    cb = cb_ref[...]
    s2 = 2.0 * jnp.sum(cb, axis=0, keepdims=True)
    q = jnp.sum(cb * cb, axis=0, keepdims=True)
    dist2 = jnp.float32(_K) * (x * x) - x * s2 + q
    dist = jnp.sqrt(jnp.maximum(dist2, 0.0))
    idx = jnp.argmin(dist, axis=1).astype(jnp.int32)
    iota_d = jax.lax.broadcasted_iota(jnp.int32, dist.shape, 1)
    oh = (iota_d == idx[:, None]).astype(jnp.bfloat16)
    cb_top = cb[:_D, :]
    cb_hi = cb_top.astype(jnp.bfloat16)
    cb_lo = (cb_top - cb_hi.astype(jnp.float32)).astype(jnp.bfloat16)
    dims = (((1,), (0,)), ((), ()))
    out_ref[...] = (
        jax.lax.dot_general(oh, cb_hi, dims, preferred_element_type=jnp.float32)
        + jax.lax.dot_general(oh, cb_lo, dims, preferred_element_type=jnp.float32))


def kernel(codes, codebook):
    b, t, d = codes.shape
    x = codes.reshape(b * t, d)
    out = pl.pallas_call(
        _vq_body,
        out_shape=jax.ShapeDtypeStruct((b * t, d), jnp.float32),
    )(x, codebook)
    return out.reshape(b, t, d)
